# baseline (speedup 1.0000x reference)
import functools

import jax
import jax.numpy as jnp
from jax import lax
from jax.experimental import pallas as pl
from jax.experimental.pallas import tpu as pltpu
from jax.experimental.pallas import tpu_sc as plsc

V = 1000000
D = 16
B = 16384
H = 200
HP = 208
H0, H1 = 104, 96
NBUF = 8
BLK = 65536

_info = plsc.get_sparse_core_info()
_NC, _NS = _info.num_cores, _info.num_subcores
NW = _NC * _NS
PW = B // NW


def _proj_body(w_ref, t_ref, o_ref):
    o_ref[...] = jnp.sum(t_ref[...] * w_ref[...], axis=0)


def _project(tab_t, w_col):
    grid = (V + BLK - 1) // BLK
    return pl.pallas_call(
        _proj_body,
        grid=(grid,),
        in_specs=[
            pl.BlockSpec((D, 1), lambda i: (0, 0)),
            pl.BlockSpec((D, BLK), lambda i: (0, i)),
        ],
        out_specs=pl.BlockSpec((BLK,), lambda i: (i,)),
        out_shape=jax.ShapeDtypeStruct((V,), jnp.float32),
    )(w_col, tab_t)


def _sc_body(x_hbm, proj_hbm, b_hbm, out_hbm,
             idx_r, vals_v, tbuf, outv, b_v, *sems):
    c = lax.axis_index("c")
    s = lax.axis_index("s")
    wid = s * _NC + c
    base = wid * PW
    sem_g = sems[:NBUF]
    sem_i = sems[NBUF:]

    pltpu.sync_copy(b_hbm, b_v)
    b_vec = b_v[...]

    zero = jnp.zeros((D,), jnp.float32)
    for slot in range(NBUF):
        vals_v[slot, pl.ds(H - 8, D)] = zero

    NI = 2 * NBUF

    def issue_idx(i, q):
        pltpu.async_copy(x_hbm.at[base + i], idx_r.at[q], sem_i[q])

    def wait_idx(q):
        pltpu.make_async_copy(x_hbm.at[0], idx_r.at[q], sem_i[q]).wait()

    def issue_gather(i, e, q):
        pltpu.async_copy(proj_hbm.at[idx_r.at[q, pl.ds(0, H0)]],
                         vals_v.at[e, pl.ds(0, H0)], sem_g[e])
        pltpu.async_copy(proj_hbm.at[idx_r.at[q, pl.ds(H0, H1)]],
                         vals_v.at[e, pl.ds(H0, H1)], sem_g[e])

    def wait_gather(e):
        pltpu.make_async_copy(proj_hbm.at[pl.ds(0, H)],
                              vals_v.at[e, pl.ds(0, H)], sem_g[e]).wait()

    for i in range(NI):
        issue_idx(i, i)
    for i in range(NBUF):
        wait_idx(i)
        issue_gather(i, i, i)

    def outer(k, carry):
        for e in range(NI):
            i = k * NI + e
            eg = e % NBUF
            wait_gather(eg)

            @pl.when(i + NI < PW)
            def _():
                issue_idx(i + NI, e)

            acc = zero
            for j in range(HP // D):
                acc = acc + vals_v[eg, pl.ds(j * D, D)]
            tbuf[pl.ds(i * D, D)] = acc

            @pl.when(i + NBUF < PW)
            def _():
                q = (e + NBUF) % NI
                wait_idx(q)
                issue_gather(i + NBUF, eg, q)
        return carry

    lax.fori_loop(0, PW // NI, outer, 0)

    def finalize(g, carry):
        row_ids = (g * D + lax.iota(jnp.int32, D)) * D
        zv = jnp.zeros((D,), jnp.float32)
        for l in range(D):
            zv = zv + plsc.load_gather(tbuf, [row_ids + l])
        z = zv / jnp.float32(H) + b_vec
        y = 1.0 / (1.0 + jnp.exp(-z))
        v = y * 10000.0
        v = (v + 8388608.0) - 8388608.0
        outv[pl.ds(g * D, D)] = v / 10000.0
        return carry

    lax.fori_loop(0, PW // D, finalize, 0)
    pltpu.sync_copy(outv, out_hbm.at[pl.ds(base, PW)])


@jax.jit
def _run(x2d, proj, b16):
    mesh = plsc.VectorSubcoreMesh(core_axis_name="c", subcore_axis_name="s")
    f = functools.partial(
        pl.kernel,
        out_type=jax.ShapeDtypeStruct((B,), jnp.float32),
        mesh=mesh,
        compiler_params=pltpu.CompilerParams(needs_layout_passes=False,
                                             use_tc_tiling_on_sc=False),
        scratch_types=[
            pltpu.VMEM((2 * NBUF, H), jnp.int32),
            pltpu.VMEM((NBUF, HP), jnp.float32),
            pltpu.VMEM((PW * D,), jnp.float32),
            pltpu.VMEM((PW,), jnp.float32),
            pltpu.VMEM((D,), jnp.float32),
        ] + [pltpu.SemaphoreType.DMA] * (3 * NBUF),
    )(_sc_body)
    return f(x2d, proj, b16)


def kernel(x, embed_table, lin_w, lin_b):
    tab_t = embed_table.T
    w_col = jnp.reshape(lin_w, (D, 1))
    b16 = jnp.broadcast_to(lin_b, (D,))
    proj = _project(tab_t, w_col)
    y = _run(x, proj, b16)
    return jnp.reshape(y, (B, 1))


# speedup vs baseline: 1.0098x; 1.0098x over previous
import functools

import jax
import jax.numpy as jnp
from jax import lax
from jax.experimental import pallas as pl
from jax.experimental.pallas import tpu as pltpu
from jax.experimental.pallas import tpu_sc as plsc

V = 1000000
D = 16
B = 16384
H = 200
HP = 208
H0, H1 = 104, 96
NBUF = 8
BLK = 131072

_info = plsc.get_sparse_core_info()
_NC, _NS = _info.num_cores, _info.num_subcores
NW = _NC * _NS
PW = B // NW


def _proj_body(w_ref, t_ref, o_ref):
    o_ref[...] = jnp.sum(t_ref[...] * w_ref[...], axis=0)


def _project(tab_t, w_col):
    grid = (V + BLK - 1) // BLK
    return pl.pallas_call(
        _proj_body,
        grid=(grid,),
        in_specs=[
            pl.BlockSpec((D, 1), lambda i: (0, 0)),
            pl.BlockSpec((D, BLK), lambda i: (0, i)),
        ],
        out_specs=pl.BlockSpec((BLK,), lambda i: (i,)),
        out_shape=jax.ShapeDtypeStruct((V,), jnp.float32),
    )(w_col, tab_t)


def _sc_body(x_hbm, proj_hbm, b_hbm, out_hbm,
             idx_r, vals_v, tbuf, outv, b_v, *sems):
    c = lax.axis_index("c")
    s = lax.axis_index("s")
    wid = s * _NC + c
    base = wid * PW
    sem_g = sems[:NBUF]
    sem_i = sems[NBUF:]

    pltpu.sync_copy(b_hbm, b_v)
    b_vec = b_v[...]

    zero = jnp.zeros((D,), jnp.float32)
    for slot in range(NBUF):
        vals_v[slot, pl.ds(H - 8, D)] = zero

    NI = 2 * NBUF

    def issue_idx(i, q):
        pltpu.async_copy(x_hbm.at[base + i], idx_r.at[q], sem_i[q])

    def wait_idx(q):
        pltpu.make_async_copy(x_hbm.at[0], idx_r.at[q], sem_i[q]).wait()

    def issue_gather(i, e, q):
        for off, ln in ((0, 56), (56, 48), (104, 48), (152, 48)):
            pltpu.async_copy(proj_hbm.at[idx_r.at[q, pl.ds(off, ln)]],
                             vals_v.at[e, pl.ds(off, ln)], sem_g[e])

    def wait_gather(e):
        pltpu.make_async_copy(proj_hbm.at[pl.ds(0, H)],
                              vals_v.at[e, pl.ds(0, H)], sem_g[e]).wait()

    for i in range(NI):
        issue_idx(i, i)
    for i in range(NBUF):
        wait_idx(i)
        issue_gather(i, i, i)

    def outer(k, carry):
        for e in range(NI):
            i = k * NI + e
            eg = e % NBUF
            wait_gather(eg)

            @pl.when(i + NI < PW)
            def _():
                issue_idx(i + NI, e)

            acc = zero
            for j in range(HP // D):
                acc = acc + vals_v[eg, pl.ds(j * D, D)]
            tbuf[pl.ds(i * D, D)] = acc

            @pl.when(i + NBUF < PW)
            def _():
                q = (e + NBUF) % NI
                wait_idx(q)
                issue_gather(i + NBUF, eg, q)
        return carry

    lax.fori_loop(0, PW // NI, outer, 0)

    def finalize(g, carry):
        row_ids = (g * D + lax.iota(jnp.int32, D)) * D
        zv = jnp.zeros((D,), jnp.float32)
        for l in range(D):
            zv = zv + plsc.load_gather(tbuf, [row_ids + l])
        z = zv / jnp.float32(H) + b_vec
        y = 1.0 / (1.0 + jnp.exp(-z))
        v = y * 10000.0
        v = (v + 8388608.0) - 8388608.0
        outv[pl.ds(g * D, D)] = v / 10000.0
        return carry

    lax.fori_loop(0, PW // D, finalize, 0)
    pltpu.sync_copy(outv, out_hbm.at[pl.ds(base, PW)])


@jax.jit
def _run(x2d, proj, b16):
    mesh = plsc.VectorSubcoreMesh(core_axis_name="c", subcore_axis_name="s")
    f = functools.partial(
        pl.kernel,
        out_type=jax.ShapeDtypeStruct((B,), jnp.float32),
        mesh=mesh,
        compiler_params=pltpu.CompilerParams(needs_layout_passes=False,
                                             use_tc_tiling_on_sc=False),
        scratch_types=[
            pltpu.VMEM((2 * NBUF, H), jnp.int32),
            pltpu.VMEM((NBUF, HP), jnp.float32),
            pltpu.VMEM((PW * D,), jnp.float32),
            pltpu.VMEM((PW,), jnp.float32),
            pltpu.VMEM((D,), jnp.float32),
        ] + [pltpu.SemaphoreType.DMA] * (3 * NBUF),
    )(_sc_body)
    return f(x2d, proj, b16)


def kernel(x, embed_table, lin_w, lin_b):
    tab_t = embed_table.T
    w_col = jnp.reshape(lin_w, (D, 1))
    b16 = jnp.broadcast_to(lin_b, (D,))
    proj = _project(tab_t, w_col)
    y = _run(x, proj, b16)
    return jnp.reshape(y, (B, 1))


# speedup vs baseline: 1.0363x; 1.0262x over previous
import functools

import jax
import jax.numpy as jnp
from jax import lax
from jax.experimental import pallas as pl
from jax.experimental.pallas import tpu as pltpu
from jax.experimental.pallas import tpu_sc as plsc

V = 1000000
D = 16
B = 16384
H = 200
HP = 208
H0, H1 = 104, 96
NBUF = 8
BLK = 131072

_info = plsc.get_sparse_core_info()
_NC, _NS = _info.num_cores, _info.num_subcores
NW = _NC * _NS
PW = B // NW


def _proj_body(w_ref, t_ref, o_ref):
    o_ref[...] = jnp.sum(t_ref[...] * w_ref[...], axis=0)


def _project(tab_t, w_col):
    grid = (V + BLK - 1) // BLK
    return pl.pallas_call(
        _proj_body,
        grid=(grid,),
        in_specs=[
            pl.BlockSpec((D, 1), lambda i: (0, 0)),
            pl.BlockSpec((D, BLK), lambda i: (0, i)),
        ],
        out_specs=pl.BlockSpec((BLK,), lambda i: (i,)),
        out_shape=jax.ShapeDtypeStruct((V,), jnp.float32),
    )(w_col, tab_t)


def _sc_body(x_hbm, proj_hbm, b_hbm, out_hbm,
             idx_r, vals_v, tbuf, outv, b_v, *sems):
    c = lax.axis_index("c")
    s = lax.axis_index("s")
    wid = s * _NC + c
    base = wid * PW
    sem_g = sems

    pltpu.sync_copy(x_hbm.at[pl.ds(base, PW)], idx_r)
    pltpu.sync_copy(b_hbm, b_v)
    b_vec = b_v[...]

    zero = jnp.zeros((D,), jnp.float32)
    for slot in range(NBUF):
        vals_v[slot, pl.ds(H - 8, D)] = zero

    def issue(i, slot):
        pltpu.async_copy(proj_hbm.at[idx_r.at[i, pl.ds(0, H0)]],
                         vals_v.at[slot, pl.ds(0, H0)], sem_g[slot])
        pltpu.async_copy(proj_hbm.at[idx_r.at[i, pl.ds(H0, H1)]],
                         vals_v.at[slot, pl.ds(H0, H1)], sem_g[slot])

    def wait(slot):
        pltpu.make_async_copy(proj_hbm.at[pl.ds(0, H)],
                              vals_v.at[slot, pl.ds(0, H)], sem_g[slot]).wait()

    for e in range(NBUF):
        issue(e, e)

    def outer(k, carry):
        for e in range(NBUF):
            i = k * NBUF + e
            wait(e)
            acc = zero
            for j in range(HP // D):
                acc = acc + vals_v[e, pl.ds(j * D, D)]
            tbuf[pl.ds(i * D, D)] = acc

            @pl.when(i + NBUF < PW)
            def _():
                issue(i + NBUF, e)
        return carry

    lax.fori_loop(0, PW // NBUF, outer, 0)

    def finalize(g, carry):
        row_ids = (g * D + lax.iota(jnp.int32, D)) * D
        zv = jnp.zeros((D,), jnp.float32)
        for l in range(D):
            zv = zv + plsc.load_gather(tbuf, [row_ids + l])
        z = zv / jnp.float32(H) + b_vec
        y = 1.0 / (1.0 + jnp.exp(-z))
        v = y * 10000.0
        v = (v + 8388608.0) - 8388608.0
        outv[pl.ds(g * D, D)] = v / 10000.0
        return carry

    lax.fori_loop(0, PW // D, finalize, 0)
    pltpu.sync_copy(outv, out_hbm.at[pl.ds(base, PW)])


@jax.jit
def _run(x2d, proj, b16):
    mesh = plsc.VectorSubcoreMesh(core_axis_name="c", subcore_axis_name="s")
    f = functools.partial(
        pl.kernel,
        out_type=jax.ShapeDtypeStruct((B,), jnp.float32),
        mesh=mesh,
        compiler_params=pltpu.CompilerParams(needs_layout_passes=False,
                                             use_tc_tiling_on_sc=False),
        scratch_types=[
            pltpu.VMEM((PW, H), jnp.int32),
            pltpu.VMEM((NBUF, HP), jnp.float32),
            pltpu.VMEM((PW * D,), jnp.float32),
            pltpu.VMEM((PW,), jnp.float32),
            pltpu.VMEM((D,), jnp.float32),
        ] + [pltpu.SemaphoreType.DMA] * NBUF,
    )(_sc_body)
    return f(x2d, proj, b16)


def kernel(x, embed_table, lin_w, lin_b):
    tab_t = embed_table.T
    w_col = jnp.reshape(lin_w, (D, 1))
    b16 = jnp.broadcast_to(lin_b, (D,))
    proj = _project(tab_t, w_col)
    y = _run(x, proj, b16)
    return jnp.reshape(y, (B, 1))
